# baseline (speedup 1.0000x reference)
import jax
import jax.numpy as jnp
from jax.experimental import pallas as pl
from jax.experimental.pallas import tpu as pltpu

_M_BLK = 512


def _ffn_kernel(x_ref, w1_ref, w3_ref, w2_ref, o_ref, inter_ref):
    i = pl.program_id(0)
    n = pl.num_programs(0)
    slot = jax.lax.rem(i, 2)
    prev_slot = jax.lax.rem(i + 1, 2)

    @pl.when(i < n - 1)
    def _up():
        xb = x_ref[...].astype(jnp.bfloat16)
        a = jnp.dot(xb, w1_ref[...], preferred_element_type=jnp.float32)
        b = jnp.dot(xb, w3_ref[...], preferred_element_type=jnp.float32)
        inter_ref[slot] = (a * jax.nn.sigmoid(a) * b).astype(jnp.bfloat16)

    @pl.when(i > 0)
    def _down():
        o_ref[...] = jnp.dot(
            inter_ref[prev_slot], w2_ref[...], preferred_element_type=jnp.float32
        )


def kernel(x, gate_W, W1, W3, W2):
    B, S, H = x.shape
    h = x.reshape(-1, H)
    M = h.shape[0]
    w1 = W1[0].astype(jnp.bfloat16)
    w3 = W3[0].astype(jnp.bfloat16)
    w2 = W2[0].astype(jnp.bfloat16)
    F = w1.shape[1]
    nblk = M // _M_BLK
    out = pl.pallas_call(
        _ffn_kernel,
        grid=(nblk + 1,),
        in_specs=[
            pl.BlockSpec((_M_BLK, H), lambda i: (jnp.minimum(i, pl.num_programs(0) - 2), 0)),
            pl.BlockSpec((H, F), lambda i: (0, 0)),
            pl.BlockSpec((H, F), lambda i: (0, 0)),
            pl.BlockSpec((F, H), lambda i: (0, 0)),
        ],
        out_specs=pl.BlockSpec(
            (_M_BLK, H), lambda i: (jnp.maximum(i - 1, 0), 0)
        ),
        out_shape=jax.ShapeDtypeStruct((M, H), jnp.float32),
        scratch_shapes=[pltpu.VMEM((2, _M_BLK, F), jnp.bfloat16)],
    )(h, w1, w3, w2)
    return out.reshape(B, S, H)


# speedup vs baseline: 1.0346x; 1.0346x over previous
import jax
import jax.numpy as jnp
from jax.experimental import pallas as pl
from jax.experimental.pallas import tpu as pltpu

_M_BLK = 512
_F_SUB = 256


def _ffn_kernel(x_ref, w1_ref, w3_ref, w2_ref, o_ref, xb_ref, inter_ref):
    xb_ref[...] = x_ref[...].astype(jnp.bfloat16)
    xb = xb_ref[...]
    F = w1_ref.shape[1]
    for f in range(F // _F_SUB):
        cols = pl.ds(f * _F_SUB, _F_SUB)
        a = jnp.dot(xb, w1_ref[:, cols], preferred_element_type=jnp.float32)
        b = jnp.dot(xb, w3_ref[:, cols], preferred_element_type=jnp.float32)
        inter_ref[:, cols] = (a * jax.nn.sigmoid(a) * b).astype(jnp.bfloat16)
    o_ref[...] = jnp.dot(
        inter_ref[...], w2_ref[...], preferred_element_type=jnp.float32
    )


def kernel(x, gate_W, W1, W3, W2):
    B, S, H = x.shape
    h = x.reshape(-1, H)
    M = h.shape[0]
    w1 = W1[0].astype(jnp.bfloat16)
    w3 = W3[0].astype(jnp.bfloat16)
    w2 = W2[0].astype(jnp.bfloat16)
    F = w1.shape[1]
    out = pl.pallas_call(
        _ffn_kernel,
        grid=(M // _M_BLK,),
        in_specs=[
            pl.BlockSpec((_M_BLK, H), lambda i: (i, 0)),
            pl.BlockSpec((H, F), lambda i: (0, 0)),
            pl.BlockSpec((H, F), lambda i: (0, 0)),
            pl.BlockSpec((F, H), lambda i: (0, 0)),
        ],
        out_specs=pl.BlockSpec((_M_BLK, H), lambda i: (i, 0)),
        out_shape=jax.ShapeDtypeStruct((M, H), jnp.float32),
        scratch_shapes=[
            pltpu.VMEM((_M_BLK, H), jnp.bfloat16),
            pltpu.VMEM((_M_BLK, F), jnp.bfloat16),
        ],
    )(h, w1, w3, w2)
    return out.reshape(B, S, H)
